# Initial kernel scaffold; baseline (speedup 1.0000x reference)
import jax
import jax.numpy as jnp
from jax.experimental import pallas as pl


def kernel(graph, feat, graph_neg, input_features_neg, input_features_nei, W1, b1, W2, b2):
    raise NotImplementedError("write your pallas kernel here")


# speedup vs baseline: 3.6726x; 3.6726x over previous
import functools

import jax
import jax.numpy as jnp
from jax import lax
from jax.experimental import pallas as pl
from jax.experimental.pallas import tpu as pltpu
from jax.experimental.pallas import tpu_sc as plsc

N = 10000
E = 320000
D = 128

NC = 2
NS = 16
L = 16

CHUNK = 128
EPC = ((E + NS * CHUNK - 1) // (NS * CHUNK)) * NS * CHUNK
PADE = EPC - E
EPT = EPC // NS
NPAD = 10240
RPT = NPAD // NS
DEG_LEN = 2 * NPAD
DPT = DEG_LEN // NS

BLK = 2000
NB = N // BLK


def _vsc_mesh():
    return plsc.VectorSubcoreMesh(core_axis_name="c", subcore_axis_name="s",
                                  num_cores=NC, num_subcores=NS)


def _deg_body(didx_hbm, out_hbm, idx_v, ones_v, zb_v, deg_sh, sem):
    c = lax.axis_index("c")
    s = lax.axis_index("s")
    for i in range(CHUNK // L):
        ones_v[pl.ds(i * L, L)] = jnp.ones((L,), jnp.float32)
        zb_v[pl.ds(i * L, L)] = jnp.zeros((L,), jnp.float32)
    for k in range(DPT // CHUNK):
        pltpu.sync_copy(zb_v, deg_sh.at[pl.ds(s * DPT + k * CHUNK, CHUNK)])
    plsc.subcore_barrier()

    tpt = 2 * EPT
    base = c * (2 * EPC) + s * tpt

    def body(j, carry):
        off = base + j * CHUNK
        pltpu.sync_copy(didx_hbm.at[pl.ds(off, CHUNK)], idx_v)
        pltpu.sync_copy(ones_v, deg_sh.at[idx_v], add=True)
        return carry

    lax.fori_loop(0, tpt // CHUNK, body, 0)
    plsc.subcore_barrier()
    for k in range(DPT // CHUNK):
        pltpu.sync_copy(deg_sh.at[pl.ds(s * DPT + k * CHUNK, CHUNK)], zb_v)
        pltpu.sync_copy(zb_v, out_hbm.at[pl.ds(c * DEG_LEN + s * DPT + k * CHUNK, CHUNK)])


@functools.cache
def _deg_call():
    return pl.kernel(
        _deg_body,
        out_type=jax.ShapeDtypeStruct((2 * DEG_LEN,), jnp.float32),
        mesh=_vsc_mesh(),
        scratch_types=[
            pltpu.VMEM((CHUNK,), jnp.int32),
            pltpu.VMEM((CHUNK,), jnp.float32),
            pltpu.VMEM((CHUNK,), jnp.float32),
            pltpu.VMEM_SHARED((DEG_LEN,), jnp.float32),
            pltpu.SemaphoreType.DMA,
        ],
    )


def _agg_body(h_hbm, gsrc_hbm, gdst_hbm, out_hbm,
              idxs_v, idxd_v, rows_v, agg_sh, sem):
    c = lax.axis_index("c")
    s = lax.axis_index("s")

    def zbody(r, carry):
        for i in range(D // L):
            rows_v[r, pl.ds(i * L, L)] = jnp.zeros((L,), jnp.float32)
        return carry

    lax.fori_loop(0, CHUNK, zbody, 0)
    for k in range(RPT // CHUNK):
        pltpu.sync_copy(rows_v, agg_sh.at[pl.ds(s * RPT + k * CHUNK, CHUNK)])
    plsc.subcore_barrier()

    base = c * EPC + s * EPT

    def body(j, carry):
        off = base + j * CHUNK
        pltpu.sync_copy(gsrc_hbm.at[pl.ds(off, CHUNK)], idxs_v)
        pltpu.sync_copy(gdst_hbm.at[pl.ds(off, CHUNK)], idxd_v)
        pltpu.async_copy(h_hbm.at[idxs_v], rows_v, sem).wait()
        pltpu.sync_copy(rows_v, agg_sh.at[idxd_v], add=True)
        return carry

    lax.fori_loop(0, EPT // CHUNK, body, 0)
    plsc.subcore_barrier()
    for k in range(RPT // CHUNK):
        pltpu.sync_copy(agg_sh.at[pl.ds(s * RPT + k * CHUNK, CHUNK)], rows_v)
        pltpu.sync_copy(rows_v, out_hbm.at[pl.ds(c * NPAD + s * RPT + k * CHUNK, CHUNK)])


@functools.cache
def _agg_call():
    return pl.kernel(
        _agg_body,
        out_type=jax.ShapeDtypeStruct((2 * NPAD, D), jnp.float32),
        mesh=_vsc_mesh(),
        scratch_types=[
            pltpu.VMEM((CHUNK,), jnp.int32),
            pltpu.VMEM((CHUNK,), jnp.int32),
            pltpu.VMEM((CHUNK, D), jnp.float32),
            pltpu.VMEM_SHARED((NPAD, D), jnp.float32),
            pltpu.SemaphoreType.DMA,
        ],
    )


def _inv_sqrt_deg(d):
    return lax.rsqrt(jnp.where(d > 0, d, 1.0))


def _prescale_body(x_ref, dego_ref, out_ref):
    out_ref[0] = x_ref[0] * _inv_sqrt_deg(dego_ref[0])


def _prescale(xs, dego):
    return pl.pallas_call(
        _prescale_body,
        grid=(2, NB),
        in_specs=[
            pl.BlockSpec((1, BLK, D), lambda c, r: (c, r, 0)),
            pl.BlockSpec((1, BLK, 1), lambda c, r: (c, r, 0)),
        ],
        out_specs=pl.BlockSpec((1, BLK, D), lambda c, r: (c, r, 0)),
        out_shape=jax.ShapeDtypeStruct((2, N, D), jnp.float32),
    )(xs, dego)


def _dense_mid_body(agg_ref, degi_ref, dego_ref, w_ref, b_ref, out_ref):
    a = agg_ref[0] * _inv_sqrt_deg(degi_ref[0])
    z = jnp.dot(a, w_ref[...], preferred_element_type=jnp.float32) + b_ref[...]
    z = jnp.maximum(z, 0.0)
    out_ref[0] = z * _inv_sqrt_deg(dego_ref[0])


def _dense_mid(agg, degi, dego, w, b):
    return pl.pallas_call(
        _dense_mid_body,
        grid=(2, NB),
        in_specs=[
            pl.BlockSpec((1, BLK, D), lambda c, r: (c, r, 0)),
            pl.BlockSpec((1, BLK, 1), lambda c, r: (c, r, 0)),
            pl.BlockSpec((1, BLK, 1), lambda c, r: (c, r, 0)),
            pl.BlockSpec((D, D), lambda c, r: (0, 0)),
            pl.BlockSpec((1, D), lambda c, r: (0, 0)),
        ],
        out_specs=pl.BlockSpec((1, BLK, D), lambda c, r: (c, r, 0)),
        out_shape=jax.ShapeDtypeStruct((2, N, D), jnp.float32),
    )(agg, degi, dego, w, b)


def _dense_out_body(agg_ref, degi_ref, w_ref, b_ref, out_ref):
    a = agg_ref[0] * _inv_sqrt_deg(degi_ref[0])
    y = jnp.dot(a, w_ref[...], preferred_element_type=jnp.float32) + b_ref[...]
    nrm = jnp.sqrt(jnp.sum(y * y, axis=-1, keepdims=True))
    out_ref[0] = y / jnp.maximum(nrm, 1e-12)


def _dense_out(agg, degi, w, b):
    return pl.pallas_call(
        _dense_out_body,
        grid=(2, NB),
        in_specs=[
            pl.BlockSpec((1, BLK, D), lambda c, r: (c, r, 0)),
            pl.BlockSpec((1, BLK, 1), lambda c, r: (c, r, 0)),
            pl.BlockSpec((D, D), lambda c, r: (0, 0)),
            pl.BlockSpec((1, D), lambda c, r: (0, 0)),
        ],
        out_specs=pl.BlockSpec((1, BLK, D), lambda c, r: (c, r, 0)),
        out_shape=jax.ShapeDtypeStruct((2, N, D), jnp.float32),
    )(agg, degi, w, b)


def kernel(graph, feat, graph_neg, input_features_neg, input_features_nei,
           W1, b1, W2, b2):
    del input_features_nei
    src0, dst0 = graph[0], graph[1]
    src1, dst1 = graph_neg[0], graph_neg[1]

    padz = jnp.zeros((PADE,), jnp.int32)
    padb = jnp.full((PADE,), N, jnp.int32)
    padb2 = jnp.full((PADE,), NPAD + N, jnp.int32)

    gsrc = jnp.concatenate([src0, padz, src1 + N, padz])
    gdst = jnp.concatenate([dst0, padb, dst1, padb])
    didx = jnp.concatenate([
        src0, padb, dst0 + NPAD, padb2,
        src1, padb, dst1 + NPAD, padb2,
    ])

    deg = _deg_call()(didx).reshape(2, 2, NPAD)
    dego = deg[:, 0, :N].reshape(2, N, 1)
    degi = deg[:, 1, :N].reshape(2, N, 1)

    xs = jnp.stack([feat, input_features_neg])
    h1 = _prescale(xs, dego)
    agg1 = _agg_call()(h1.reshape(2 * N, D), gsrc, gdst).reshape(2, NPAD, D)
    h2 = _dense_mid(agg1, degi, dego, W1, b1.reshape(1, D))
    agg2 = _agg_call()(h2.reshape(2 * N, D), gsrc, gdst).reshape(2, NPAD, D)
    out = _dense_out(agg2, degi, W2, b2.reshape(1, D))
    return out[0], out[1]
